# baseline (speedup 1.0000x reference)
import jax
import jax.numpy as jnp
from jax import lax
from jax.experimental import pallas as pl
from jax.experimental.pallas import tpu as pltpu
from jax.experimental.pallas import tpu_sc as plsc

BATCH = 16384
EMBED_DIM = 64
NUM_CORES = 2
NUM_SUBCORES = 16
NUM_TILES = NUM_CORES * NUM_SUBCORES
B_PER_TILE = BATCH // NUM_TILES
CHUNK = 128
NCHUNK = B_PER_TILE // CHUNK

_MESH = plsc.VectorSubcoreMesh(core_axis_name="c", subcore_axis_name="s")


def _gather_body(uidx_hbm, iidx_hbm, uemb_hbm, iemb_hbm, u_hbm, v_hbm,
                 uidx_v, iidx_v, urows_v, irows_v, sem):
    wid = lax.axis_index("s") * NUM_CORES + lax.axis_index("c")
    base = wid * B_PER_TILE
    pltpu.sync_copy(uidx_hbm.at[wid], uidx_v)
    pltpu.sync_copy(iidx_hbm.at[wid], iidx_v)
    copies = []
    for j in range(NCHUNK):
        copies.append(pltpu.async_copy(
            uemb_hbm.at[uidx_v.at[j]], urows_v.at[pl.ds(j * CHUNK, CHUNK)], sem))
        copies.append(pltpu.async_copy(
            iemb_hbm.at[iidx_v.at[j]], irows_v.at[pl.ds(j * CHUNK, CHUNK)], sem))
    for c in copies:
        c.wait()
    pltpu.sync_copy(urows_v, u_hbm.at[pl.ds(base, B_PER_TILE)])
    pltpu.sync_copy(irows_v, v_hbm.at[pl.ds(base, B_PER_TILE)])


def _sc_gather(user_idx, item_idx, user_emb, item_emb):
    k = pl.kernel(
        _gather_body,
        out_type=(
            jax.ShapeDtypeStruct((BATCH, EMBED_DIM), jnp.float32),
            jax.ShapeDtypeStruct((BATCH, EMBED_DIM), jnp.float32),
        ),
        mesh=_MESH,
        compiler_params=pltpu.CompilerParams(use_tc_tiling_on_sc=False),
        scratch_types=[
            pltpu.VMEM((NCHUNK, CHUNK), jnp.int32),
            pltpu.VMEM((NCHUNK, CHUNK), jnp.int32),
            pltpu.VMEM((B_PER_TILE, EMBED_DIM), jnp.float32),
            pltpu.VMEM((B_PER_TILE, EMBED_DIM), jnp.float32),
            pltpu.SemaphoreType.DMA,
        ],
    )
    uidx = user_idx.astype(jnp.int32).reshape(NUM_TILES, NCHUNK, CHUNK)
    iidx = item_idx.astype(jnp.int32).reshape(NUM_TILES, NCHUNK, CHUNK)
    return k(uidx, iidx, user_emb, item_emb)


BLK = 2048


def _mlp_body(u_ref, v_ref, w1a_ref, w1b_ref, b1_ref, w2_ref, b2_ref,
              w3_ref, b3_ref, wp_ref, bp_ref, out_ref):
    f32 = jnp.float32
    h = jnp.dot(u_ref[...], w1a_ref[...], preferred_element_type=f32)
    h += jnp.dot(v_ref[...], w1b_ref[...], preferred_element_type=f32)
    h = jnp.maximum(h + b1_ref[...], 0.0)
    h = jnp.maximum(jnp.dot(h, w2_ref[...], preferred_element_type=f32)
                    + b2_ref[...], 0.0)
    h = jnp.maximum(jnp.dot(h, w3_ref[...], preferred_element_type=f32)
                    + b3_ref[...], 0.0)
    p = jnp.sum(h * wp_ref[...], axis=1) + bp_ref[0, 0]
    out_ref[...] = jax.nn.sigmoid(p)


def _tc_mlp(u, v, W1, b1, W2, b2, W3, b3, Wp, bp):
    w1a = W1[:, :EMBED_DIM].T
    w1b = W1[:, EMBED_DIM:].T
    w2 = W2.T
    w3 = W3.T
    grid = (BATCH // BLK,)
    full = lambda shape: pl.BlockSpec(shape, lambda i: (0,) * len(shape))
    return pl.pallas_call(
        _mlp_body,
        grid=grid,
        in_specs=[
            pl.BlockSpec((BLK, EMBED_DIM), lambda i: (i, 0)),
            pl.BlockSpec((BLK, EMBED_DIM), lambda i: (i, 0)),
            full((EMBED_DIM, 128)),
            full((EMBED_DIM, 128)),
            full((1, 128)),
            full((128, EMBED_DIM)),
            full((1, EMBED_DIM)),
            full((EMBED_DIM, 32)),
            full((1, 32)),
            full((1, 32)),
            full((1, 1)),
        ],
        out_specs=pl.BlockSpec((BLK,), lambda i: (i,)),
        out_shape=jax.ShapeDtypeStruct((BATCH,), jnp.float32),
    )(u, v, w1a, w1b, b1.reshape(1, -1), w2, b2.reshape(1, -1),
      w3, b3.reshape(1, -1), Wp, bp.reshape(1, 1))


def kernel(user_indices, item_indices, user_emb, item_emb,
           W1, b1, W2, b2, W3, b3, Wp, bp):
    u, v = _sc_gather(user_indices, item_indices, user_emb, item_emb)
    return _tc_mlp(u, v, W1, b1, W2, b2, W3, b3, Wp, bp)
